# baseline (speedup 1.0000x reference)
import jax
import jax.numpy as jnp
from jax.experimental import pallas as pl
from jax.experimental.pallas import tpu as pltpu

_EPS = 1e-5
_B, _N = 16, 2048
_RING = 4
_TROWS = 512
_NTILES = 4 + 12


def _dgt(x, w):
    return jax.lax.dot_general(x, w, (((1,), (1,)), ((), ())),
                               preferred_element_type=jnp.float32)


def _body(xp_ref, w1t_ref, w2t_ref, wft_ref, wgt_ref, crow_ref, w4t_ref,
          b4_ref, Wl1_ref, Wl2_ref, Wl3_ref, Wl4_ref, bl1_ref,
          bl2_ref, bl3_ref, bl4_ref, o_ref,
          f_db, g_db, z_acc, s_w1, s_w2, ring,
          sem_w1, sem_w2, sem_r):
    pid = pl.program_id(0)
    par = jax.lax.rem(pid, 2)
    bpar = jax.lax.rem(pid + 1, 2)

    def tile(j):
        if j < 4:
            return Wl3_ref.at[pl.ds(_TROWS * j, _TROWS), :]
        return Wl4_ref.at[pl.ds(_TROWS * (j - 4), _TROWS), :]

    @pl.when(pid == 0)
    def _():
        pltpu.make_async_copy(Wl1_ref, s_w1, sem_w1).start()

    @pl.when(pid == 0)
    def _():
        pltpu.make_async_copy(Wl2_ref, s_w2, sem_w2).start()

    for k in range(_RING):
        @pl.when(pid == 1 + (k // 2))
        def _(k=k):
            pltpu.make_async_copy(tile(k), ring.at[k], sem_r.at[k]).start()

    for s in range(4):
        x = xp_ref[s]
        f1 = jnp.maximum(jnp.dot(x, w1t_ref[...],
                                 preferred_element_type=jnp.float32),
                         0.0).astype(jnp.bfloat16)
        fraw = jnp.dot(f1, w2t_ref[...],
                       preferred_element_type=jnp.float32)
        fb16 = fraw.astype(jnp.bfloat16)
        f_db[par, s] = fb16
        g = jnp.max(fb16, axis=0, keepdims=True)
        g_db[par, s] = jnp.broadcast_to(g, (8, 256))

    for s in range(4):
        fb = f_db[bpar, s]
        g8 = g_db[bpar, s]
        grow = (jnp.dot(g8, wgt_ref[...],
                        preferred_element_type=jnp.float32)[0:1]
                + crow_ref[...])
        h1 = jnp.maximum(jnp.dot(fb, wft_ref[...],
                                 preferred_element_type=jnp.float32)
                         + grow, 0.0).astype(jnp.bfloat16)
        h = jnp.dot(h1, w4t_ref[...],
                    preferred_element_type=jnp.float32)
        zrow = jnp.max(h, axis=0, keepdims=True) + b4_ref[...]
        z_acc[pl.ds(jnp.maximum(pid - 1, 0) * 4 + s, 1), :] = zrow

    @pl.when(pid == _B // 4)
    def _():
        z = z_acc[...]
        pltpu.make_async_copy(Wl1_ref, s_w1, sem_w1).wait()
        d1 = jnp.maximum(_dgt(z, s_w1[...]) + bl1_ref[...], 0.0)

        pltpu.make_async_copy(Wl2_ref, s_w2, sem_w2).wait()
        d2 = jnp.maximum(_dgt(d1, s_w2[...]) + bl2_ref[...], 0.0)

        d3parts = []
        d3 = None
        for j in range(_NTILES):
            s = j % _RING
            pltpu.make_async_copy(tile(j), ring.at[s], sem_r.at[s]).wait()
            if j < 4:
                d3parts.append(jnp.maximum(
                    _dgt(d2, ring[s])
                    + bl3_ref[:, _TROWS * j:_TROWS * (j + 1)], 0.0))
                if j == 3:
                    d3 = jnp.concatenate(d3parts, axis=1)
            else:
                jo = j - 4
                pj = (_dgt(d3, ring[s])
                      + bl4_ref[:, _TROWS * jo:_TROWS * (jo + 1)])
                o_ref[:, _TROWS * jo:_TROWS * (jo + 1)] = pj
            if j + _RING < _NTILES:
                pltpu.make_async_copy(tile(j + _RING), ring.at[s],
                                      sem_r.at[s]).start()


def _fused(xp, w1t, w2t, wft, wgt, crow, w4t, b4,
           Wl1, Wl2, Wl3, Wl4, bl1, bl2, bl3, bl4):
    rep = lambda shape: pl.BlockSpec(shape, lambda i: (0,) * len(shape))
    hbm = pl.BlockSpec(memory_space=pl.ANY)
    return pl.pallas_call(
        _body,
        grid=(_B // 4 + 1,),
        in_specs=[
            pl.BlockSpec((4, _N, 8),
                         lambda i: (jnp.minimum(i, _B // 4 - 1), 0, 0)),
            rep(w1t.shape), rep(w2t.shape),
            rep(wft.shape), rep(wgt.shape), rep(crow.shape), rep(w4t.shape),
            rep(b4.shape),
            hbm, hbm, hbm, hbm,
            rep(bl1.shape), rep(bl2.shape), rep(bl3.shape), rep(bl4.shape),
        ],
        out_specs=pl.BlockSpec((_B, 6144), lambda i: (0, 0)),
        out_shape=jax.ShapeDtypeStruct((_B, 6144), jnp.float32),
        scratch_shapes=[
            pltpu.VMEM((2, 4, _N, 256), jnp.bfloat16),
            pltpu.VMEM((2, 4, 8, 256), jnp.bfloat16),
            pltpu.VMEM((_B, 1024), jnp.float32),
            pltpu.VMEM((2048, 1024), jnp.float32),
            pltpu.VMEM((2048, 2048), jnp.float32),
            pltpu.VMEM((_RING, _TROWS, 2048), jnp.float32),
            pltpu.SemaphoreType.DMA,
            pltpu.SemaphoreType.DMA,
            pltpu.SemaphoreType.DMA((_RING,)),
        ],
    )(xp, w1t, w2t, wft, wgt, crow, w4t, b4,
      Wl1, Wl2, Wl3, Wl4, bl1, bl2, bl3, bl4)


def kernel(partial, W_c1, b_c1, bn1_g, bn1_b, W_c2, b_c2, W_c3, b_c3, bn2_g,
           bn2_b, W_c4, b_c4, W_l1, b_l1, W_l2, b_l2, W_l3, b_l3, W_l4, b_l4):
    B, N, _ = partial.shape
    bf = jnp.bfloat16

    s1 = bn1_g / jnp.sqrt(1.0 + _EPS)
    s2 = bn2_g / jnp.sqrt(1.0 + _EPS)
    w1 = W_c1 * s1[:, None]
    b1 = b_c1 * s1 + bn1_b
    w3 = W_c3 * s2[:, None]
    b3 = b_c3 * s2 + bn2_b

    ones = jnp.ones((B, N, 1), jnp.float32)
    zeros = jnp.zeros((B, N, 4), jnp.float32)
    xp = jnp.concatenate([partial, ones, zeros], axis=-1).astype(bf)
    w1t = jnp.concatenate([w1.T, b1[None, :],
                           jnp.zeros((4, 128), jnp.float32)], axis=0)
    wgt = w3[:, :256].T
    wft = w3[:, 256:].T
    crow = (b_c2 @ (wgt + wft) + b3).reshape(1, -1)

    r = lambda b: b.reshape(1, -1)
    d = _fused(xp, w1t.astype(bf), W_c2.T.astype(bf), wft.astype(bf),
               wgt.astype(bf), crow, W_c4.T.astype(bf), r(b_c4),
               W_l1, W_l2, W_l3, W_l4,
               r(b_l1), r(b_l2), r(b_l3), r(b_l4))
    return d.reshape(B, -1, 3)


# speedup vs baseline: 1.0262x; 1.0262x over previous
import jax
import jax.numpy as jnp
from jax.experimental import pallas as pl
from jax.experimental.pallas import tpu as pltpu

_EPS = 1e-5
_B, _N = 16, 2048
_RING = 2
_TROWS = 512
_NTILES = 4 + 12


def _dgt(x, w):
    return jax.lax.dot_general(x, w, (((1,), (1,)), ((), ())),
                               preferred_element_type=jnp.float32)


def _body(xp_ref, w1t_ref, w2t_ref, wft_ref, wgt_ref, crow_ref, w4t_ref,
          b4_ref, Wl1_ref, Wl2_ref, Wl3_ref, Wl4_ref, bl1_ref,
          bl2_ref, bl3_ref, bl4_ref, o_ref,
          f_db, g_db, z_acc, s_w1, b_w4, ring,
          sem_w1, sem_r):
    pid = pl.program_id(0)
    par = jax.lax.rem(pid, 2)
    bpar = jax.lax.rem(pid + 1, 2)

    def w4tile(q):
        return Wl4_ref.at[pl.ds(_TROWS * q, _TROWS), :]

    def w2tile(q):
        return Wl2_ref.at[pl.ds(_TROWS * q, _TROWS), :]

    def w3tile(q):
        return Wl3_ref.at[pl.ds(_TROWS * q, _TROWS), :]

    @pl.when(pid == 0)
    def _():
        pltpu.make_async_copy(Wl1_ref, s_w1, sem_w1).start()

    @pl.when(pid == 1)
    def _():
        for k in range(_RING):
            pltpu.make_async_copy(w4tile(k), ring.at[k], sem_r.at[k]).start()

    for p in range(2, 8):
        @pl.when(pid == p)
        def _(p=p):
            for q in (2 * (p - 2), 2 * (p - 2) + 1):
                sl = q % _RING
                pltpu.make_async_copy(w4tile(q), ring.at[sl],
                                      sem_r.at[sl]).wait()
                b_w4[pl.ds(_TROWS * q, _TROWS), :] = (
                    ring[sl].astype(jnp.bfloat16))
                nq = q + _RING
                if nq < 12:
                    pltpu.make_async_copy(w4tile(nq), ring.at[sl],
                                          sem_r.at[sl]).start()
                else:
                    pltpu.make_async_copy(w2tile(nq - 12), ring.at[sl],
                                          sem_r.at[sl]).start()

    for s in range(2):
        x = xp_ref[s]
        f1 = jnp.maximum(jnp.dot(x, w1t_ref[...],
                                 preferred_element_type=jnp.float32),
                         0.0).astype(jnp.bfloat16)
        fraw = jnp.dot(f1, w2t_ref[...],
                       preferred_element_type=jnp.float32)
        fb16 = fraw.astype(jnp.bfloat16)
        f_db[par, s] = fb16
        g = jnp.max(fb16, axis=0, keepdims=True)
        g_db[par, s] = jnp.broadcast_to(g, (8, 256))

    for s in range(2):
        fb = f_db[bpar, s]
        g8 = g_db[bpar, s]
        grow = (jnp.dot(g8, wgt_ref[...],
                        preferred_element_type=jnp.float32)[0:1]
                + crow_ref[...])
        h1 = jnp.maximum(jnp.dot(fb, wft_ref[...],
                                 preferred_element_type=jnp.float32)
                         + grow, 0.0).astype(jnp.bfloat16)
        h = jnp.dot(h1, w4t_ref[...],
                    preferred_element_type=jnp.float32)
        zrow = jnp.max(h, axis=0, keepdims=True) + b4_ref[...]
        z_acc[pl.ds(jnp.maximum(pid - 1, 0) * 2 + s, 1), :] = zrow

    @pl.when(pid == _B // 2)
    def _():
        z = z_acc[...]
        pltpu.make_async_copy(Wl1_ref, s_w1, sem_w1).wait()
        d1 = jnp.maximum(_dgt(z, s_w1[...]) + bl1_ref[...], 0.0)

        d2parts = []
        for j in range(4):
            sl = j % _RING
            pltpu.make_async_copy(w2tile(j), ring.at[sl], sem_r.at[sl]).wait()
            d2parts.append(jnp.maximum(
                _dgt(d1, ring[sl])
                + bl2_ref[:, _TROWS * j:_TROWS * (j + 1)], 0.0))
            if j + _RING < 4:
                pltpu.make_async_copy(w2tile(j + _RING), ring.at[sl],
                                      sem_r.at[sl]).start()
            else:
                pltpu.make_async_copy(w3tile(j + _RING - 4), ring.at[sl],
                                      sem_r.at[sl]).start()
        d2 = jnp.concatenate(d2parts, axis=1)

        d3parts = []
        for j in range(4):
            sl = j % _RING
            pltpu.make_async_copy(w3tile(j), ring.at[sl], sem_r.at[sl]).wait()
            d3parts.append(jnp.maximum(
                _dgt(d2, ring[sl])
                + bl3_ref[:, _TROWS * j:_TROWS * (j + 1)], 0.0))
            if j + _RING < 4:
                pltpu.make_async_copy(w3tile(j + _RING), ring.at[sl],
                                      sem_r.at[sl]).start()
        d3 = jnp.concatenate(d3parts, axis=1).astype(jnp.bfloat16)

        for j in range(12):
            pj = (_dgt(d3, b_w4[pl.ds(_TROWS * j, _TROWS), :])
                  + bl4_ref[:, _TROWS * j:_TROWS * (j + 1)])
            o_ref[:, _TROWS * j:_TROWS * (j + 1)] = pj


def _fused(xp, w1t, w2t, wft, wgt, crow, w4t, b4,
           Wl1, Wl2, Wl3, Wl4, bl1, bl2, bl3, bl4):
    rep = lambda shape: pl.BlockSpec(shape, lambda i: (0,) * len(shape))
    hbm = pl.BlockSpec(memory_space=pl.ANY)
    return pl.pallas_call(
        _body,
        grid=(_B // 2 + 1,),
        in_specs=[
            pl.BlockSpec((2, _N, 8),
                         lambda i: (jnp.minimum(i, _B // 2 - 1), 0, 0)),
            rep(w1t.shape), rep(w2t.shape),
            rep(wft.shape), rep(wgt.shape), rep(crow.shape), rep(w4t.shape),
            rep(b4.shape),
            hbm, hbm, hbm, hbm,
            rep(bl1.shape), rep(bl2.shape), rep(bl3.shape), rep(bl4.shape),
        ],
        out_specs=pl.BlockSpec((_B, 6144), lambda i: (0, 0)),
        out_shape=jax.ShapeDtypeStruct((_B, 6144), jnp.float32),
        scratch_shapes=[
            pltpu.VMEM((2, 2, _N, 256), jnp.bfloat16),
            pltpu.VMEM((2, 2, 8, 256), jnp.bfloat16),
            pltpu.VMEM((_B, 1024), jnp.float32),
            pltpu.VMEM((2048, 1024), jnp.float32),
            pltpu.VMEM((6144, 2048), jnp.bfloat16),
            pltpu.VMEM((_RING, _TROWS, 2048), jnp.float32),
            pltpu.SemaphoreType.DMA,
            pltpu.SemaphoreType.DMA((_RING,)),
        ],
    )(xp, w1t, w2t, wft, wgt, crow, w4t, b4,
      Wl1, Wl2, Wl3, Wl4, bl1, bl2, bl3, bl4)


def kernel(partial, W_c1, b_c1, bn1_g, bn1_b, W_c2, b_c2, W_c3, b_c3, bn2_g,
           bn2_b, W_c4, b_c4, W_l1, b_l1, W_l2, b_l2, W_l3, b_l3, W_l4, b_l4):
    B, N, _ = partial.shape
    bf = jnp.bfloat16

    s1 = bn1_g / jnp.sqrt(1.0 + _EPS)
    s2 = bn2_g / jnp.sqrt(1.0 + _EPS)
    w1 = W_c1 * s1[:, None]
    b1 = b_c1 * s1 + bn1_b
    w3 = W_c3 * s2[:, None]
    b3 = b_c3 * s2 + bn2_b

    ones = jnp.ones((B, N, 1), jnp.float32)
    zeros = jnp.zeros((B, N, 4), jnp.float32)
    xp = jnp.concatenate([partial, ones, zeros], axis=-1).astype(bf)
    w1t = jnp.concatenate([w1.T, b1[None, :],
                           jnp.zeros((4, 128), jnp.float32)], axis=0)
    wgt = w3[:, :256].T
    wft = w3[:, 256:].T
    crow = (b_c2 @ (wgt + wft) + b3).reshape(1, -1)

    r = lambda b: b.reshape(1, -1)
    d = _fused(xp, w1t.astype(bf), W_c2.T.astype(bf), wft.astype(bf),
               wgt.astype(bf), crow, W_c4.T.astype(bf), r(b_c4),
               W_l1, W_l2, W_l3, W_l4,
               r(b_l1), r(b_l2), r(b_l3), r(b_l4))
    return d.reshape(B, -1, 3)


# speedup vs baseline: 1.0532x; 1.0264x over previous
import jax
import jax.numpy as jnp
from jax.experimental import pallas as pl
from jax.experimental.pallas import tpu as pltpu

_EPS = 1e-5
_B, _N = 16, 2048
_RING = 6
_TROWS = 512
_NTILES = 4 + 12


def _dgt(x, w):
    return jax.lax.dot_general(x, w, (((1,), (1,)), ((), ())),
                               preferred_element_type=jnp.float32)


def _body(xp_ref, w1t_ref, w2t_ref, wft_ref, wgt_ref, crow_ref, w4t_ref,
          b4_ref, Wl1_ref, Wl2_ref, Wl3_ref, Wl4_ref, bl1_ref,
          bl2_ref, bl3_ref, bl4_ref, o_ref,
          f_db, g_db, z_acc, s_w1, s_w2, ring,
          sem_w1, sem_w2, sem_r):
    pid = pl.program_id(0)
    par = jax.lax.rem(pid, 2)
    bpar = jax.lax.rem(pid + 1, 2)

    def tile(j):
        if j < 4:
            return Wl3_ref.at[pl.ds(_TROWS * j, _TROWS), :]
        return Wl4_ref.at[pl.ds(_TROWS * (j - 4), _TROWS), :]

    @pl.when(pid == 0)
    def _():
        pltpu.make_async_copy(Wl1_ref, s_w1, sem_w1).start()

    @pl.when(pid == 1)
    def _():
        pltpu.make_async_copy(Wl2_ref, s_w2, sem_w2).start()

    for k in range(_RING):
        @pl.when(pid == jnp.minimum(1 + k, 7))
        def _(k=k):
            pltpu.make_async_copy(tile(k), ring.at[k], sem_r.at[k]).start()

    for s in range(2):
        x = xp_ref[s]
        f1 = jnp.maximum(jnp.dot(x, w1t_ref[...],
                                 preferred_element_type=jnp.float32),
                         0.0).astype(jnp.bfloat16)
        fraw = jnp.dot(f1, w2t_ref[...],
                       preferred_element_type=jnp.float32)
        fb16 = fraw.astype(jnp.bfloat16)
        f_db[par, s] = fb16
        g = jnp.max(fb16, axis=0, keepdims=True)
        g_db[par, s] = jnp.broadcast_to(g, (8, 256))

    for s in range(2):
        fb = f_db[bpar, s]
        g8 = g_db[bpar, s]
        grow = (jnp.dot(g8, wgt_ref[...],
                        preferred_element_type=jnp.float32)[0:1]
                + crow_ref[...])
        h1 = jnp.maximum(jnp.dot(fb, wft_ref[...],
                                 preferred_element_type=jnp.float32)
                         + grow, 0.0).astype(jnp.bfloat16)
        h = jnp.dot(h1, w4t_ref[...],
                    preferred_element_type=jnp.float32)
        zrow = jnp.max(h, axis=0, keepdims=True) + b4_ref[...]
        z_acc[pl.ds(jnp.maximum(pid - 1, 0) * 2 + s, 1), :] = zrow

    @pl.when(pid == _B // 2)
    def _():
        z = z_acc[...]
        pltpu.make_async_copy(Wl1_ref, s_w1, sem_w1).wait()
        d1 = jnp.maximum(_dgt(z, s_w1[...]) + bl1_ref[...], 0.0)

        pltpu.make_async_copy(Wl2_ref, s_w2, sem_w2).wait()
        d2 = jnp.maximum(_dgt(d1, s_w2[...]) + bl2_ref[...], 0.0)

        d3parts = []
        d3 = None
        for j in range(_NTILES):
            s = j % _RING
            pltpu.make_async_copy(tile(j), ring.at[s], sem_r.at[s]).wait()
            if j < 4:
                d3parts.append(jnp.maximum(
                    _dgt(d2, ring[s])
                    + bl3_ref[:, _TROWS * j:_TROWS * (j + 1)], 0.0))
                if j == 3:
                    d3 = jnp.concatenate(d3parts, axis=1)
            else:
                jo = j - 4
                pj = (_dgt(d3, ring[s])
                      + bl4_ref[:, _TROWS * jo:_TROWS * (jo + 1)])
                o_ref[:, _TROWS * jo:_TROWS * (jo + 1)] = pj
            if j + _RING < _NTILES:
                pltpu.make_async_copy(tile(j + _RING), ring.at[s],
                                      sem_r.at[s]).start()


def _fused(xp, w1t, w2t, wft, wgt, crow, w4t, b4,
           Wl1, Wl2, Wl3, Wl4, bl1, bl2, bl3, bl4):
    rep = lambda shape: pl.BlockSpec(shape, lambda i: (0,) * len(shape))
    hbm = pl.BlockSpec(memory_space=pl.ANY)
    return pl.pallas_call(
        _body,
        grid=(_B // 2 + 1,),
        in_specs=[
            pl.BlockSpec((2, _N, 8),
                         lambda i: (jnp.minimum(i, _B // 2 - 1), 0, 0)),
            rep(w1t.shape), rep(w2t.shape),
            rep(wft.shape), rep(wgt.shape), rep(crow.shape), rep(w4t.shape),
            rep(b4.shape),
            hbm, hbm, hbm, hbm,
            rep(bl1.shape), rep(bl2.shape), rep(bl3.shape), rep(bl4.shape),
        ],
        out_specs=pl.BlockSpec((_B, 6144), lambda i: (0, 0)),
        out_shape=jax.ShapeDtypeStruct((_B, 6144), jnp.float32),
        scratch_shapes=[
            pltpu.VMEM((2, 2, _N, 256), jnp.bfloat16),
            pltpu.VMEM((2, 2, 8, 256), jnp.bfloat16),
            pltpu.VMEM((_B, 1024), jnp.float32),
            pltpu.VMEM((2048, 1024), jnp.float32),
            pltpu.VMEM((2048, 2048), jnp.float32),
            pltpu.VMEM((_RING, _TROWS, 2048), jnp.float32),
            pltpu.SemaphoreType.DMA,
            pltpu.SemaphoreType.DMA,
            pltpu.SemaphoreType.DMA((_RING,)),
        ],
        compiler_params=pltpu.CompilerParams(
            vmem_limit_bytes=63 * 1024 * 1024),
    )(xp, w1t, w2t, wft, wgt, crow, w4t, b4,
      Wl1, Wl2, Wl3, Wl4, bl1, bl2, bl3, bl4)


def kernel(partial, W_c1, b_c1, bn1_g, bn1_b, W_c2, b_c2, W_c3, b_c3, bn2_g,
           bn2_b, W_c4, b_c4, W_l1, b_l1, W_l2, b_l2, W_l3, b_l3, W_l4, b_l4):
    B, N, _ = partial.shape
    bf = jnp.bfloat16

    s1 = bn1_g / jnp.sqrt(1.0 + _EPS)
    s2 = bn2_g / jnp.sqrt(1.0 + _EPS)
    w1 = W_c1 * s1[:, None]
    b1 = b_c1 * s1 + bn1_b
    w3 = W_c3 * s2[:, None]
    b3 = b_c3 * s2 + bn2_b

    ones = jnp.ones((B, N, 1), jnp.float32)
    zeros = jnp.zeros((B, N, 4), jnp.float32)
    xp = jnp.concatenate([partial, ones, zeros], axis=-1).astype(bf)
    w1t = jnp.concatenate([w1.T, b1[None, :],
                           jnp.zeros((4, 128), jnp.float32)], axis=0)
    wgt = w3[:, :256].T
    wft = w3[:, 256:].T
    crow = (b_c2 @ (wgt + wft) + b3).reshape(1, -1)

    r = lambda b: b.reshape(1, -1)
    d = _fused(xp, w1t.astype(bf), W_c2.T.astype(bf), wft.astype(bf),
               wgt.astype(bf), crow, W_c4.T.astype(bf), r(b_c4),
               W_l1, W_l2, W_l3, W_l4,
               r(b_l1), r(b_l2), r(b_l3), r(b_l4))
    return d.reshape(B, -1, 3)
